# Initial kernel scaffold; baseline (speedup 1.0000x reference)
import jax
import jax.numpy as jnp
from jax.experimental import pallas as pl


def kernel(x, adj, w0, b0, w1, b1, w2, b2, g0, be0, m0, v0, g1, be1, m1, v1):
    raise NotImplementedError("write your pallas kernel here")


# speedup vs baseline: 16.7817x; 16.7817x over previous
import jax
import jax.numpy as jnp
from jax.experimental import pallas as pl
from jax.experimental.pallas import tpu as pltpu

NEG = -1e30
_BJORCK_ITERS = 5


def _bjorck_kernel(w_ref, s_ref, o_ref):
    w = w_ref[0]
    for _ in range(_BJORCK_ITERS):
        wtw = jax.lax.dot_general(w, w, (((0,), (0,)), ((), ())),
                                  preferred_element_type=jnp.float32)
        w = 1.5 * w - 0.5 * jnp.dot(w, wtw,
                                    preferred_element_type=jnp.float32)
    o_ref[0] = w * s_ref[0]


def _orthonormalize_scaled(w_stack, scale_stack):
    nl, dpad, dout = w_stack.shape
    return pl.pallas_call(
        _bjorck_kernel,
        out_shape=jax.ShapeDtypeStruct((nl, dpad, dout), jnp.float32),
        grid=(nl,),
        in_specs=[
            pl.BlockSpec((1, dpad, dout), lambda i: (i, 0, 0)),
            pl.BlockSpec((1, 1, dout), lambda i: (i, 0, 0)),
        ],
        out_specs=pl.BlockSpec((1, dpad, dout), lambda i: (i, 0, 0)),
        compiler_params=pltpu.CompilerParams(
            dimension_semantics=("parallel",)),
    )(w_stack, scale_stack)


def _xw_kernel(x_ref, w_ref, o_ref):
    o_ref[...] = jnp.dot(x_ref[...], w_ref[...],
                         preferred_element_type=jnp.float32
                         ).astype(jnp.bfloat16)


def _support(x, wf, din, dout, row0, *, row_block=1024):
    n = x.shape[0]
    rb = min(row_block, n)
    wr = row0 // din
    return pl.pallas_call(
        _xw_kernel,
        out_shape=jax.ShapeDtypeStruct((n, dout), jnp.bfloat16),
        grid=(n // rb,),
        in_specs=[
            pl.BlockSpec((rb, din), lambda i: (i, 0)),
            pl.BlockSpec((din, dout), lambda i, wr=wr: (wr, 0)),
        ],
        out_specs=pl.BlockSpec((rb, dout), lambda i: (i, 0)),
        compiler_params=pltpu.CompilerParams(
            dimension_semantics=("parallel",)),
    )(x, wf)


def _adj1_kernel(a_ref, s_ref, b_ref, o_ref, abf_ref):
    a = a_ref[...].astype(jnp.bfloat16)
    abf_ref[...] = a
    acc = jnp.dot(a, s_ref[...], preferred_element_type=jnp.float32)
    o_ref[...] = jnp.maximum(acc + b_ref[...], 0.0)


def _adj_relu_kernel(a_ref, s_ref, b_ref, o_ref):
    acc = jnp.dot(a_ref[...], s_ref[...],
                  preferred_element_type=jnp.float32)
    o_ref[...] = jnp.maximum(acc + b_ref[...], 0.0)


def _adj_lsm_kernel(a_ref, s_ref, b_ref, o_ref):
    acc = jnp.dot(a_ref[...], s_ref[...],
                  preferred_element_type=jnp.float32)
    h = acc + b_ref[...]
    m = jnp.max(h, axis=-1, keepdims=True)
    z = h - m
    lse = jnp.log(jnp.sum(jnp.exp(z), axis=-1, keepdims=True))
    o_ref[...] = z - lse


def _adj_call(body, a, support, bias, out_shapes, out_specs, *, row_block):
    n, dout = support.shape
    rb = min(row_block, n)
    return pl.pallas_call(
        body,
        out_shape=out_shapes,
        grid=(n // rb,),
        in_specs=[
            pl.BlockSpec((rb, n), lambda i: (i, 0)),
            pl.BlockSpec((n, dout), lambda i: (0, 0)),
            pl.BlockSpec((1, dout), lambda i: (0, 0)),
        ],
        out_specs=out_specs,
        compiler_params=pltpu.CompilerParams(
            dimension_semantics=("parallel",)),
    )(a, support, bias)


def _adj1(a, support, bias, *, row_block=512):
    n, dout = support.shape
    rb = min(row_block, n)
    return _adj_call(
        _adj1_kernel, a, support, bias,
        [jax.ShapeDtypeStruct((n, dout), jnp.float32),
         jax.ShapeDtypeStruct((n, n), jnp.bfloat16)],
        [pl.BlockSpec((rb, dout), lambda i: (i, 0)),
         pl.BlockSpec((rb, n), lambda i: (i, 0))],
        row_block=rb)


def _adj_relu(a, support, bias, *, row_block=512):
    n, dout = support.shape
    rb = min(row_block, n)
    return _adj_call(
        _adj_relu_kernel, a, support, bias,
        jax.ShapeDtypeStruct((n, dout), jnp.float32),
        pl.BlockSpec((rb, dout), lambda i: (i, 0)),
        row_block=rb)


def _adj_lsm(a, support, bias, *, row_block=512):
    n, dout = support.shape
    rb = min(row_block, n)
    return _adj_call(
        _adj_lsm_kernel, a, support, bias,
        jax.ShapeDtypeStruct((n, dout), jnp.float32),
        pl.BlockSpec((rb, dout), lambda i: (i, 0)),
        row_block=rb)


def kernel(x, adj, w0, b0, w1, b1, w2, b2,
           g0, be0, m0, v0, g1, be1, m1, v1):
    f32 = jnp.float32
    nfeat, nhid = w0.shape
    nclass = w2.shape[1]
    ncls_p = 128

    s0 = g0 * jax.lax.rsqrt(v0 + 1e-5)
    s1 = g1 * jax.lax.rsqrt(v1 + 1e-5)
    bf0 = ((b0 - m0) * s0 + be0).reshape(1, nhid).astype(f32)
    bf1 = ((b1 - m1) * s1 + be1).reshape(1, nhid).astype(f32)
    b2p = jnp.pad(b2, (0, ncls_p - nclass),
                  constant_values=NEG).reshape(1, ncls_p).astype(f32)

    dpad = nfeat
    w_stack = jnp.stack([
        w0,
        jnp.pad(w1, ((0, dpad - nhid), (0, 0))),
        jnp.pad(w2, ((0, dpad - nhid), (0, nhid - nclass))),
    ])
    scale_stack = jnp.stack([s0.reshape(1, nhid), s1.reshape(1, nhid),
                             jnp.ones((1, nhid), f32)])
    wf = _orthonormalize_scaled(w_stack, scale_stack).reshape(3 * dpad, nhid)

    sup1 = _support(x, wf, nfeat, nhid, 0)
    h1, adj_bf = _adj1(adj, sup1, bf0)
    sup2 = _support(h1, wf, nhid, nhid, dpad)
    h2 = _adj_relu(adj_bf, sup2, bf1)
    sup3 = _support(h2, wf, nhid, ncls_p, 2 * dpad)
    out = _adj_lsm(adj_bf, sup3, b2p)
    return out[:, :nclass]


# speedup vs baseline: 20.7928x; 1.2390x over previous
import jax
import jax.numpy as jnp
from jax.experimental import pallas as pl
from jax.experimental.pallas import tpu as pltpu

NEG = -1e30
_ITERS = 5

_XT = 1024
_AT = 128
_RT = 512

_N_X = 4
_N_A = 32
_N_R = 8

_S1 = 1 + _N_X
_S2 = _S1 + _N_A
_L2 = _S2 + 1
_S3 = _L2 + _N_R
_L3 = _S3 + 1
_STEPS = _L3 + _N_R


def _mega_kernel(x_ref, a_ref, w_ref, sc_ref, b0_ref, b1_ref, b2_ref,
                 o_ref, wf_s, a_s, s1_s, h1_s, h2_s):
    s = pl.program_id(0)
    bf16 = jnp.bfloat16

    @pl.when(s == 0)
    def _bjorck():
        for l in range(3):
            w = w_ref[l]
            for _ in range(_ITERS):
                wtw = jax.lax.dot_general(
                    w, w, (((0,), (0,)), ((), ())),
                    preferred_element_type=jnp.float32)
                w = 1.5 * w - 0.5 * jnp.dot(
                    w, wtw, preferred_element_type=jnp.float32)
            wf_s[l] = w * sc_ref[l]

    @pl.when((s >= 1) & (s < _S1))
    def _sup1():
        r = (s - 1) * _XT
        s1_s[pl.ds(r, _XT), :] = jnp.dot(
            x_ref[...], wf_s[0],
            preferred_element_type=jnp.float32).astype(bf16)

    @pl.when((s >= _S1) & (s < _S2))
    def _layer1():
        r = (s - _S1) * _AT
        a = a_ref[...].astype(bf16)
        a_s[pl.ds(r, _AT), :] = a
        acc = jnp.dot(a, s1_s[...], preferred_element_type=jnp.float32)
        h1_s[pl.ds(r, _AT), :] = jnp.maximum(
            acc + b0_ref[...], 0.0).astype(bf16)

    @pl.when(s == _S2)
    def _sup2():
        w1 = wf_s[1][:256, :].astype(bf16)
        s1_s[...] = jnp.dot(
            h1_s[...], w1, preferred_element_type=jnp.float32).astype(bf16)

    @pl.when((s >= _L2) & (s < _S3))
    def _layer2():
        r = (s - _L2) * _RT
        acc = jnp.dot(a_s[pl.ds(r, _RT), :], s1_s[...],
                      preferred_element_type=jnp.float32)
        h2_s[pl.ds(r, _RT), :] = jnp.maximum(
            acc + b1_ref[...], 0.0).astype(bf16)

    @pl.when(s == _S3)
    def _sup3():
        w2 = wf_s[2][:256, :128].astype(bf16)
        h1_s[:, :128] = jnp.dot(
            h2_s[...], w2, preferred_element_type=jnp.float32).astype(bf16)

    @pl.when(s >= _L3)
    def _layer3():
        r = (s - _L3) * _RT
        acc = jnp.dot(a_s[pl.ds(r, _RT), :], h1_s[:, :128],
                      preferred_element_type=jnp.float32)
        h = acc + b2_ref[...]
        m = jnp.max(h, axis=-1, keepdims=True)
        z = h - m
        lse = jnp.log(jnp.sum(jnp.exp(z), axis=-1, keepdims=True))
        o_ref[...] = z - lse


def kernel(x, adj, w0, b0, w1, b1, w2, b2,
           g0, be0, m0, v0, g1, be1, m1, v1):
    f32 = jnp.float32
    n = x.shape[0]
    nfeat, nhid = w0.shape
    nclass = w2.shape[1]
    ncls_p = 128

    s0 = g0 * jax.lax.rsqrt(v0 + 1e-5)
    s1 = g1 * jax.lax.rsqrt(v1 + 1e-5)
    bf0 = ((b0 - m0) * s0 + be0).reshape(1, nhid).astype(f32)
    bf1 = ((b1 - m1) * s1 + be1).reshape(1, nhid).astype(f32)
    b2p = jnp.pad(b2, (0, ncls_p - nclass),
                  constant_values=NEG).reshape(1, ncls_p).astype(f32)

    w_stack = jnp.stack([
        w0,
        jnp.pad(w1, ((0, nfeat - nhid), (0, 0))),
        jnp.pad(w2, ((0, nfeat - nhid), (0, nhid - nclass))),
    ])
    scale_stack = jnp.stack([s0.reshape(1, nhid), s1.reshape(1, nhid),
                             jnp.ones((1, nhid), f32)])

    out = pl.pallas_call(
        _mega_kernel,
        out_shape=jax.ShapeDtypeStruct((n, ncls_p), f32),
        grid=(_STEPS,),
        in_specs=[
            pl.BlockSpec((_XT, nfeat),
                         lambda i: (jnp.clip(i - 1, 0, _N_X - 1), 0)),
            pl.BlockSpec((_AT, n),
                         lambda i: (jnp.clip(i - _S1, 0, _N_A - 1), 0)),
            pl.BlockSpec((3, nfeat, nhid), lambda i: (0, 0, 0)),
            pl.BlockSpec((3, 1, nhid), lambda i: (0, 0, 0)),
            pl.BlockSpec((1, nhid), lambda i: (0, 0)),
            pl.BlockSpec((1, nhid), lambda i: (0, 0)),
            pl.BlockSpec((1, ncls_p), lambda i: (0, 0)),
        ],
        out_specs=pl.BlockSpec((_RT, ncls_p),
                               lambda i: (jnp.clip(i - _L3, 0, _N_R - 1), 0)),
        scratch_shapes=[
            pltpu.VMEM((3, nfeat, nhid), f32),
            pltpu.VMEM((n, n), jnp.bfloat16),
            pltpu.VMEM((n, nhid), jnp.bfloat16),
            pltpu.VMEM((n, nhid), jnp.bfloat16),
            pltpu.VMEM((n, nhid), jnp.bfloat16),
        ],
        compiler_params=pltpu.CompilerParams(
            dimension_semantics=("arbitrary",)),
    )(x, adj, w_stack, scale_stack, bf0, bf1, b2p)
    return out[:, :nclass]


# speedup vs baseline: 25.7516x; 1.2385x over previous
import jax
import jax.numpy as jnp
from jax.experimental import pallas as pl
from jax.experimental.pallas import tpu as pltpu

NEG = -1e30
_ITERS = 5

_XT = 1024
_AT = 256
_RT = 1024

_N_X = 4
_N_A = 16
_N_R = 4

_S1 = 1 + _N_X
_L2 = _S1 + _N_A
_L3 = _L2 + _N_R
_STEPS = _L3 + _N_R


def _bjorck(w):
    for _ in range(_ITERS):
        wtw = jax.lax.dot_general(w, w, (((0,), (0,)), ((), ())),
                                  preferred_element_type=jnp.float32)
        w = 1.5 * w - 0.5 * jnp.dot(w, wtw,
                                    preferred_element_type=jnp.float32)
    return w


def _mega_kernel(x_ref, a_ref, w0_ref, w1_ref, w2_ref,
                 b0_ref, b1_ref, b2_ref, sc0_ref, sc1_ref,
                 o_ref, wf0_s, wf1_s, wf2_s, a_s, s1_s, h1_s, h2_s):
    s = pl.program_id(0)
    bf16 = jnp.bfloat16

    @pl.when(s == 0)
    def _bj0():
        wf0_s[...] = _bjorck(w0_ref[...]) * sc0_ref[...]

    @pl.when((s >= 1) & (s < _S1))
    def _sup1():
        r = (s - 1) * _XT
        s1_s[pl.ds(r, _XT), :] = jnp.dot(
            x_ref[...], wf0_s[...],
            preferred_element_type=jnp.float32).astype(bf16)

    @pl.when(s == _S1)
    def _bj1():
        wf1_s[...] = (_bjorck(w1_ref[...]) * sc1_ref[...]).astype(bf16)

    @pl.when(s == _S1 + 1)
    def _bj2():
        wf2_s[...] = _bjorck(w2_ref[...]).astype(bf16)

    @pl.when((s >= _S1) & (s < _L2))
    def _layer1():
        r = (s - _S1) * _AT
        a = a_ref[...].astype(bf16)
        a_s[pl.ds(r, _AT), :] = a
        acc = jnp.dot(a, s1_s[...], preferred_element_type=jnp.float32)
        h1_s[pl.ds(r, _AT), :] = jnp.maximum(
            acc + b0_ref[...], 0.0).astype(bf16)

    @pl.when(s == _L2)
    def _sup2():
        s1_s[...] = jnp.dot(h1_s[...], wf1_s[...],
                            preferred_element_type=jnp.float32).astype(bf16)

    @pl.when((s >= _L2) & (s < _L3))
    def _layer2():
        r = (s - _L2) * _RT
        acc = jnp.dot(a_s[pl.ds(r, _RT), :], s1_s[...],
                      preferred_element_type=jnp.float32)
        h2_s[pl.ds(r, _RT), :] = jnp.maximum(
            acc + b1_ref[...], 0.0).astype(bf16)

    @pl.when(s == _L3)
    def _sup3():
        h1_s[:, :128] = jnp.dot(h2_s[...], wf2_s[...],
                                preferred_element_type=jnp.float32
                                ).astype(bf16)

    @pl.when(s >= _L3)
    def _layer3():
        r = (s - _L3) * _RT
        acc = jnp.dot(a_s[pl.ds(r, _RT), :], h1_s[:, :128],
                      preferred_element_type=jnp.float32)
        h = acc + b2_ref[...]
        m = jnp.max(h, axis=-1, keepdims=True)
        z = h - m
        lse = jnp.log(jnp.sum(jnp.exp(z), axis=-1, keepdims=True))
        o_ref[...] = z - lse


def kernel(x, adj, w0, b0, w1, b1, w2, b2,
           g0, be0, m0, v0, g1, be1, m1, v1):
    f32 = jnp.float32
    n = x.shape[0]
    nfeat, nhid = w0.shape
    nclass = w2.shape[1]
    ncls_p = 128

    s0 = g0 * jax.lax.rsqrt(v0 + 1e-5)
    s1 = g1 * jax.lax.rsqrt(v1 + 1e-5)
    bf0 = ((b0 - m0) * s0 + be0).reshape(1, nhid).astype(f32)
    bf1 = ((b1 - m1) * s1 + be1).reshape(1, nhid).astype(f32)
    b2p = jnp.pad(b2, (0, ncls_p - nclass),
                  constant_values=NEG).reshape(1, ncls_p).astype(f32)
    w2p = jnp.pad(w2, ((0, 0), (0, ncls_p - nclass)))

    rst = lambda i: (0, 0)
    out = pl.pallas_call(
        _mega_kernel,
        out_shape=jax.ShapeDtypeStruct((n, ncls_p), f32),
        grid=(_STEPS,),
        in_specs=[
            pl.BlockSpec((_XT, nfeat),
                         lambda i: (jnp.clip(i - 1, 0, _N_X - 1), 0)),
            pl.BlockSpec((_AT, n),
                         lambda i: (jnp.clip(i - _S1, 0, _N_A - 1), 0)),
            pl.BlockSpec((nfeat, nhid), rst),
            pl.BlockSpec((nhid, nhid), rst),
            pl.BlockSpec((nhid, ncls_p), rst),
            pl.BlockSpec((1, nhid), rst),
            pl.BlockSpec((1, nhid), rst),
            pl.BlockSpec((1, ncls_p), rst),
            pl.BlockSpec((1, nhid), rst),
            pl.BlockSpec((1, nhid), rst),
        ],
        out_specs=pl.BlockSpec((_RT, ncls_p),
                               lambda i: (jnp.clip(i - _L3, 0, _N_R - 1), 0)),
        scratch_shapes=[
            pltpu.VMEM((nfeat, nhid), f32),
            pltpu.VMEM((nhid, nhid), jnp.bfloat16),
            pltpu.VMEM((nhid, ncls_p), jnp.bfloat16),
            pltpu.VMEM((n, n), jnp.bfloat16),
            pltpu.VMEM((n, nhid), jnp.bfloat16),
            pltpu.VMEM((n, nhid), jnp.bfloat16),
            pltpu.VMEM((n, nhid), jnp.bfloat16),
        ],
        compiler_params=pltpu.CompilerParams(
            dimension_semantics=("arbitrary",)),
    )(x, adj, w0, w1, w2p, bf0, bf1, b2p,
      s0.reshape(1, nhid), s1.reshape(1, nhid))
    return out[:, :nclass]
